# baseline (speedup 1.0000x reference)
import functools

import jax
import jax.numpy as jnp
from jax import lax
from jax.experimental import pallas as pl
from jax.experimental.pallas import tpu as pltpu, tpu_sc as plsc

N = 10000
E = 320000
D = 128
H = 16
C = 5
CP = 8

NPAD = 10240
NC = 2
NS = 16
NW = NC * NS
EW = E // NW
CHUNKS = (2504, 2504, 2504, 2488)
OFFS = (0, 2504, 5008, 7512)
NITER = len(CHUNKS)
CMAX = CHUNKS[0]
RPT = NPAD // NS


def _fill(ref, n, val):
    v = jnp.full((16,), val, jnp.float32)

    def body(i, c):
        ref[pl.ds(i * 16, 16)] = v
        return c

    lax.fori_loop(0, n // 16, body, 0)


def _make_deg_kernel():
    mesh = plsc.VectorSubcoreMesh(core_axis_name="c", subcore_axis_name="s")

    @functools.partial(
        pl.kernel,
        mesh=mesh,
        out_type=jax.ShapeDtypeStruct((NC, NPAD), jnp.float32),
        scratch_types=[
            pltpu.VMEM((EW,), jnp.int32),
            pltpu.VMEM((EW,), jnp.float32),
            pltpu.VMEM((RPT,), jnp.float32),
            pltpu.VMEM_SHARED((NPAD,), jnp.float32),
            pltpu.SemaphoreType.DMA,
        ],
    )
    def deg_kernel(ei, out, dst_v, ones_v, z_v, acc, sem):
        cid = lax.axis_index("c")
        sid = lax.axis_index("s")
        wid = cid * NS + sid
        base = wid * EW

        idx_cp = pltpu.async_copy(ei.at[pl.ds(E + base, EW)], dst_v, sem)
        _fill(z_v, RPT, 0.0)
        _fill(ones_v, EW, 1.0)
        pltpu.sync_copy(z_v, acc.at[pl.ds(sid * RPT, RPT)])
        plsc.subcore_barrier()
        idx_cp.wait()
        pltpu.sync_copy(ones_v, acc.at[dst_v], add=True)
        plsc.subcore_barrier()
        pltpu.sync_copy(
            acc.at[pl.ds(sid * RPT, RPT)],
            out.at[cid, pl.ds(sid * RPT, RPT)],
        )

    return deg_kernel


EC = (1672, 1672, 1672, 1672, 1672, 1640)
ECOFF = (0, 1672, 3344, 5016, 6688, 8360)
ECN = len(EC)
ECMAX = EC[0]
DC = 5000
EPT = E // NS


def _newton_rsqrt(d):
    y = 1.0 / d
    for _ in range(22):
        y = y * (1.5 - 0.5 * d * y * y)
    return y


def _make_fused1_kernel():
    mesh = plsc.VectorSubcoreMesh(core_axis_name="c", subcore_axis_name="s")

    @functools.partial(
        pl.kernel,
        mesh=mesh,
        out_type=(
            jax.ShapeDtypeStruct((NC, NPAD), jnp.float32),
            jax.ShapeDtypeStruct((NC, NPAD, H), jnp.float32),
        ),
        scratch_types=[
            [pltpu.VMEM((DC,), jnp.int32) for _ in range(2)],
            pltpu.VMEM((DC,), jnp.float32),
            pltpu.VMEM((RPT,), jnp.float32),
            pltpu.VMEM((RPT,), jnp.float32),
            pltpu.VMEM((RPT,), jnp.float32),
            pltpu.VMEM((RPT, H), jnp.float32),
            [pltpu.VMEM((c,), jnp.int32) for c in EC],
            [pltpu.VMEM((c,), jnp.int32) for c in EC],
            [pltpu.VMEM((ECMAX, H), jnp.float32) for _ in range(2)],
            pltpu.VMEM_SHARED((NPAD,), jnp.float32),
            pltpu.VMEM_SHARED((NPAD, H), jnp.float32),
            pltpu.VMEM_SHARED((NPAD, H), jnp.float32),
            pltpu.SemaphoreType.DMA,
            pltpu.SemaphoreType.DMA,
            [pltpu.SemaphoreType.DMA for _ in range(2)],
            [pltpu.SemaphoreType.DMA for _ in range(2)],
            [pltpu.SemaphoreType.DMA for _ in range(2)],
        ],
        compiler_params=pltpu.CompilerParams(use_tc_tiling_on_sc=False),
    )
    def fused1(ei, h1, zeros, dinv_out, p1_out, dbufs, ones_v, z_v, deg_v,
               dinv_v, h1_v, srcs, dsts, rows, dacc, g1sh, acc, semi, semh,
               semd, semg, sems):
        cid = lax.axis_index("c")
        sid = lax.axis_index("s")
        wid = cid * NS + sid
        base = wid * EW
        dbase = E + sid * EPT
        rslc = pl.ds(sid * RPT, RPT)

        h1_cp = pltpu.async_copy(h1.at[rslc], h1_v, semh)
        dcp = [None, None]
        dcp[0] = pltpu.async_copy(ei.at[pl.ds(dbase, DC)], dbufs[0], semd[0])
        cps = [pltpu.async_copy(zeros.at[rslc], acc.at[rslc], semi)]
        for j in range(ECN):
            off = base + ECOFF[j]
            cps.append(pltpu.async_copy(ei.at[pl.ds(off, EC[j])],
                                        srcs[j], semi))
            cps.append(pltpu.async_copy(ei.at[pl.ds(E + off, EC[j])],
                                        dsts[j], semi))
        _fill(z_v, RPT, 0.0)
        _fill(ones_v, DC, 1.0)
        pltpu.sync_copy(z_v, dacc.at[rslc])
        plsc.subcore_barrier()

        for r in range(4):
            dcp[r % 2].wait()
            if r + 1 < 4:
                dcp[(r + 1) % 2] = pltpu.async_copy(
                    ei.at[pl.ds(dbase + (r + 1) * DC, DC)],
                    dbufs[(r + 1) % 2], semd[(r + 1) % 2])
            pltpu.sync_copy(ones_v, dacc.at[dbufs[r % 2]], add=True)
        plsc.subcore_barrier()

        pltpu.sync_copy(dacc.at[rslc], deg_v)
        h1_cp.wait()

        def nbody(k, c):
            d = deg_v[pl.ds(k * 16, 16)] + 1.0
            dinv_v[pl.ds(k * 16, 16)] = _newton_rsqrt(d)
            return c

        lax.fori_loop(0, RPT // 16, nbody, 0)

        for k in range(RPT // 16):
            dv16 = dinv_v[pl.ds(k * 16, 16)]
            for j in range(16):
                h1_v[k * 16 + j, :] = dv16[j] * h1_v[k * 16 + j, :]
        pltpu.sync_copy(h1_v, g1sh.at[rslc])
        pltpu.sync_copy(dinv_v, dinv_out.at[cid, rslc])
        for cp in cps:
            cp.wait()
        plsc.subcore_barrier()

        def buf(j):
            b = rows[j % 2]
            return b if EC[j] == ECMAX else b.at[pl.ds(0, EC[j])]

        gathers = [None, None]
        gathers[0] = pltpu.async_copy(g1sh.at[srcs[0]], buf(0), semg[0])
        scat = [None, None]
        for j in range(ECN):
            gathers[j % 2].wait()
            if j + 1 < ECN:
                nb = (j + 1) % 2
                if scat[nb] is not None:
                    scat[nb].wait()
                    scat[nb] = None
                gathers[nb] = pltpu.async_copy(
                    g1sh.at[srcs[j + 1]], buf(j + 1), semg[nb])
            scat[j % 2] = pltpu.async_copy(
                buf(j), acc.at[dsts[j]], sems[j % 2], add=True)
        for sc in scat:
            if sc is not None:
                sc.wait()
        plsc.subcore_barrier()
        pltpu.sync_copy(acc.at[rslc], p1_out.at[cid, rslc])

    return fused1


def _make_edge_kernel(width):
    mesh = plsc.VectorSubcoreMesh(core_axis_name="c", subcore_axis_name="s")

    @functools.partial(
        pl.kernel,
        mesh=mesh,
        out_type=jax.ShapeDtypeStruct((NC, NPAD, width), jnp.float32),
        scratch_types=[
            [pltpu.VMEM((c,), jnp.int32) for c in CHUNKS],
            [pltpu.VMEM((c,), jnp.int32) for c in CHUNKS],
            [pltpu.VMEM((CMAX, width), jnp.float32) for _ in range(2)],
            pltpu.VMEM_SHARED((NPAD, width), jnp.float32),
            pltpu.SemaphoreType.DMA,
            [pltpu.SemaphoreType.DMA for _ in range(2)],
            [pltpu.SemaphoreType.DMA for _ in range(2)],
        ],
        compiler_params=pltpu.CompilerParams(use_tc_tiling_on_sc=False),
    )
    def edge_kernel(ei, g, zeros, out, srcs, dsts, rows, acc, semi, semg,
                    sems):
        cid = lax.axis_index("c")
        sid = lax.axis_index("s")
        wid = cid * NS + sid
        base = wid * EW

        src0_cp = pltpu.async_copy(
            ei.at[pl.ds(base, CHUNKS[0])], srcs[0], semi)
        cps = [pltpu.async_copy(
            zeros.at[pl.ds(sid * RPT, RPT)],
            acc.at[pl.ds(sid * RPT, RPT)], semi)]
        for j in range(NITER):
            off = base + OFFS[j]
            if j > 0:
                cps.append(pltpu.async_copy(ei.at[pl.ds(off, CHUNKS[j])],
                                            srcs[j], semi))
            cps.append(pltpu.async_copy(ei.at[pl.ds(E + off, CHUNKS[j])],
                                        dsts[j], semi))

        def buf(j):
            b = rows[j % 2]
            return b if CHUNKS[j] == CMAX else b.at[pl.ds(0, CHUNKS[j])]

        src0_cp.wait()
        gathers = [None, None]
        gathers[0] = pltpu.async_copy(g.at[srcs[0]], buf(0), semg[0])
        for cp in cps:
            cp.wait()
        plsc.subcore_barrier()

        scat = [None, None]
        for j in range(NITER):
            gathers[j % 2].wait()
            if j + 1 < NITER:
                nb = (j + 1) % 2
                if scat[nb] is not None:
                    scat[nb].wait()
                    scat[nb] = None
                gathers[nb] = pltpu.async_copy(
                    g.at[srcs[j + 1]], buf(j + 1), semg[nb])
            scat[j % 2] = pltpu.async_copy(
                buf(j), acc.at[dsts[j]], sems[j % 2], add=True)
        for sc in scat:
            if sc is not None:
                sc.wait()

        plsc.subcore_barrier()
        pltpu.sync_copy(
            acc.at[pl.ds(sid * RPT, RPT)],
            out.at[cid, pl.ds(sid * RPT, RPT)],
        )

    return edge_kernel


def _k2a_body(x_ref, w1_ref, h1_ref):
    h1_ref[pl.ds(0, N), :] = jnp.dot(x_ref[...], w1_ref[...],
                                     preferred_element_type=jnp.float32)
    h1_ref[pl.ds(N, NPAD - N), :] = jnp.zeros((NPAD - N, H), jnp.float32)


def _k6_body(q_ref, g2_ref, dinv_ref, w2_ref, b2_ref, w3_ref, b3_ref,
             w4_ref, b4_ref, out_ref):
    dinv = dinv_ref[...]
    s2 = dinv * (q_ref[0:N, :] + q_ref[NPAD:NPAD + N, :] + g2_ref[...])
    w23 = jnp.dot(w2_ref[...], w3_ref[...],
                  preferred_element_type=jnp.float32)
    b23 = jnp.dot(b2_ref[...], w3_ref[...],
                  preferred_element_type=jnp.float32) + b3_ref[...]
    z1 = jnp.maximum(
        jnp.dot(s2, w23, preferred_element_type=jnp.float32) + b23, 0.0)
    z = jnp.dot(z1, w4_ref[...], preferred_element_type=jnp.float32) \
        + b4_ref[...]
    m = jnp.max(z, axis=1, keepdims=True)
    lse = jnp.log(jnp.sum(jnp.exp(z - m), axis=1, keepdims=True)) + m
    out_ref[...] = z - lse


def kernel(x, edge_index, W1, b1, W2, b2, W3, b3, W4, b4):
    fused1 = _make_fused1_kernel()
    edge16 = _make_edge_kernel(H)

    ei_flat = edge_index.reshape(2 * E)

    h1 = pl.pallas_call(
        _k2a_body,
        out_shape=jax.ShapeDtypeStruct((NPAD, H), jnp.float32),
    )(x, W1)

    z16 = jnp.zeros((NPAD, H), jnp.float32)
    dinvp, p1 = fused1(ei_flat, h1, z16)

    dinv = dinvp[0, :N][:, None]
    g1 = h1[:N] * dinv

    r1 = jnp.maximum(dinv * (p1[0, :N] + p1[1, :N] + g1) + b1[None, :], 0.0)
    g2 = r1 * dinv

    p2 = edge16(ei_flat, g2, z16).reshape(NC * NPAD, H)

    out = pl.pallas_call(
        _k6_body,
        out_shape=jax.ShapeDtypeStruct((N, C), jnp.float32),
    )(p2, g2, dinv, W2, b2.reshape(1, C), W3, b3.reshape(1, 32), W4,
      b4.reshape(1, C))
    return out


# speedup vs baseline: 1.0103x; 1.0103x over previous
import functools

import jax
import jax.numpy as jnp
from jax import lax
from jax.experimental import pallas as pl
from jax.experimental.pallas import tpu as pltpu, tpu_sc as plsc

N = 10000
E = 320000
D = 128
H = 16
C = 5
CP = 8

NPAD = 10240
NC = 2
NS = 16
NW = NC * NS
EW = E // NW
CHUNKS = (2504, 2504, 2504, 2488)
OFFS = (0, 2504, 5008, 7512)
NITER = len(CHUNKS)
CMAX = CHUNKS[0]
RPT = NPAD // NS


def _fill(ref, n, val):
    v = jnp.full((16,), val, jnp.float32)

    def body(i, c):
        ref[pl.ds(i * 16, 16)] = v
        return c

    lax.fori_loop(0, n // 16, body, 0)


def _make_deg_kernel():
    mesh = plsc.VectorSubcoreMesh(core_axis_name="c", subcore_axis_name="s")

    @functools.partial(
        pl.kernel,
        mesh=mesh,
        out_type=jax.ShapeDtypeStruct((NC, NPAD), jnp.float32),
        scratch_types=[
            pltpu.VMEM((EW,), jnp.int32),
            pltpu.VMEM((EW,), jnp.float32),
            pltpu.VMEM((RPT,), jnp.float32),
            pltpu.VMEM_SHARED((NPAD,), jnp.float32),
            pltpu.SemaphoreType.DMA,
        ],
    )
    def deg_kernel(ei, out, dst_v, ones_v, z_v, acc, sem):
        cid = lax.axis_index("c")
        sid = lax.axis_index("s")
        wid = cid * NS + sid
        base = wid * EW

        idx_cp = pltpu.async_copy(ei.at[pl.ds(E + base, EW)], dst_v, sem)
        _fill(z_v, RPT, 0.0)
        _fill(ones_v, EW, 1.0)
        pltpu.sync_copy(z_v, acc.at[pl.ds(sid * RPT, RPT)])
        plsc.subcore_barrier()
        idx_cp.wait()
        pltpu.sync_copy(ones_v, acc.at[dst_v], add=True)
        plsc.subcore_barrier()
        pltpu.sync_copy(
            acc.at[pl.ds(sid * RPT, RPT)],
            out.at[cid, pl.ds(sid * RPT, RPT)],
        )

    return deg_kernel


def _make_edge_kernel(width):
    mesh = plsc.VectorSubcoreMesh(core_axis_name="c", subcore_axis_name="s")

    @functools.partial(
        pl.kernel,
        mesh=mesh,
        out_type=jax.ShapeDtypeStruct((NC, NPAD, width), jnp.float32),
        scratch_types=[
            [pltpu.VMEM((c,), jnp.int32) for c in CHUNKS],
            [pltpu.VMEM((c,), jnp.int32) for c in CHUNKS],
            [pltpu.VMEM((CMAX, width), jnp.float32) for _ in range(2)],
            pltpu.VMEM_SHARED((NPAD, width), jnp.float32),
            pltpu.SemaphoreType.DMA,
            [pltpu.SemaphoreType.DMA for _ in range(2)],
            [pltpu.SemaphoreType.DMA for _ in range(2)],
        ],
        compiler_params=pltpu.CompilerParams(use_tc_tiling_on_sc=False),
    )
    def edge_kernel(ei, g, zeros, out, srcs, dsts, rows, acc, semi, semg,
                    sems):
        cid = lax.axis_index("c")
        sid = lax.axis_index("s")
        wid = cid * NS + sid
        base = wid * EW

        src0_cp = pltpu.async_copy(
            ei.at[pl.ds(base, CHUNKS[0])], srcs[0], semi)
        cps = [pltpu.async_copy(
            zeros.at[pl.ds(sid * RPT, RPT)],
            acc.at[pl.ds(sid * RPT, RPT)], semi)]
        for j in range(NITER):
            off = base + OFFS[j]
            if j > 0:
                cps.append(pltpu.async_copy(ei.at[pl.ds(off, CHUNKS[j])],
                                            srcs[j], semi))
            cps.append(pltpu.async_copy(ei.at[pl.ds(E + off, CHUNKS[j])],
                                        dsts[j], semi))

        def buf(j):
            b = rows[j % 2]
            return b if CHUNKS[j] == CMAX else b.at[pl.ds(0, CHUNKS[j])]

        src0_cp.wait()
        gathers = [None, None]
        gathers[0] = pltpu.async_copy(g.at[srcs[0]], buf(0), semg[0])
        for cp in cps:
            cp.wait()
        plsc.subcore_barrier()

        scat = [None, None]
        for j in range(NITER):
            gathers[j % 2].wait()
            if j + 1 < NITER:
                nb = (j + 1) % 2
                if scat[nb] is not None:
                    scat[nb].wait()
                    scat[nb] = None
                gathers[nb] = pltpu.async_copy(
                    g.at[srcs[j + 1]], buf(j + 1), semg[nb])
            scat[j % 2] = pltpu.async_copy(
                buf(j), acc.at[dsts[j]], sems[j % 2], add=True)
        for sc in scat:
            if sc is not None:
                sc.wait()

        plsc.subcore_barrier()
        pltpu.sync_copy(
            acc.at[pl.ds(sid * RPT, RPT)],
            out.at[cid, pl.ds(sid * RPT, RPT)],
        )

    return edge_kernel


def _k2a_body(x_ref, w1_ref, h1_ref):
    h1_ref[...] = jnp.dot(x_ref[...], w1_ref[...],
                          preferred_element_type=jnp.float32)


def _k6_body(q_ref, g2_ref, dinv_ref, w2_ref, b2_ref, w3_ref, b3_ref,
             w4_ref, b4_ref, out_ref):
    dinv = dinv_ref[...]
    s2 = dinv * (q_ref[0:N, :] + q_ref[NPAD:NPAD + N, :] + g2_ref[...])
    w23 = jnp.dot(w2_ref[...], w3_ref[...],
                  preferred_element_type=jnp.float32)
    b23 = jnp.dot(b2_ref[...], w3_ref[...],
                  preferred_element_type=jnp.float32) + b3_ref[...]
    z1 = jnp.maximum(
        jnp.dot(s2, w23, preferred_element_type=jnp.float32) + b23, 0.0)
    z = jnp.dot(z1, w4_ref[...], preferred_element_type=jnp.float32) \
        + b4_ref[...]
    m = jnp.max(z, axis=1, keepdims=True)
    lse = jnp.log(jnp.sum(jnp.exp(z - m), axis=1, keepdims=True)) + m
    out_ref[...] = z - lse


def kernel(x, edge_index, W1, b1, W2, b2, W3, b3, W4, b4):
    deg_k = _make_deg_kernel()
    edge16 = _make_edge_kernel(H)

    ei_flat = edge_index.reshape(2 * E)

    h1 = pl.pallas_call(
        _k2a_body,
        out_shape=jax.ShapeDtypeStruct((N, H), jnp.float32),
    )(x, W1)

    degp = deg_k(ei_flat)

    deg = degp[0, :N] + degp[1, :N] + 1.0
    dinv = lax.rsqrt(deg)[:, None]
    g1 = h1 * dinv

    z16 = jnp.zeros((NPAD, H), jnp.float32)
    p1 = edge16(ei_flat, g1, z16)

    r1 = jnp.maximum(dinv * (p1[0, :N] + p1[1, :N] + g1) + b1[None, :], 0.0)
    g2 = r1 * dinv

    p2 = edge16(ei_flat, g2, z16).reshape(NC * NPAD, H)

    out = pl.pallas_call(
        _k6_body,
        out_shape=jax.ShapeDtypeStruct((N, C), jnp.float32),
    )(p2, g2, dinv, W2, b2.reshape(1, C), W3, b3.reshape(1, 32), W4,
      b4.reshape(1, C))
    return out
